# baseline (speedup 1.0000x reference)
import jax
import jax.numpy as jnp
from jax import lax
from jax.experimental import pallas as pl
from jax.experimental.pallas import tpu as pltpu
from jax.experimental.pallas import tpu_sc as plsc

N = 32768
NS = 16
L = 16
CHUNK = N // NS
HALF = CHUNK // 2
SUBH = HALF // L

_mesh = plsc.VectorSubcoreMesh(
    core_axis_name="c", subcore_axis_name="s", num_cores=1
)


def _excl(t):
    return plsc.cumsum(t) - t


def _sc_cumsum_body(x_hbm, out_hbm, x_v, tot_v, all_v, shared,
                    sem_in0, sem_in1, sem_out0):
    sid = lax.axis_index("s")
    base = sid * CHUNK

    in0 = pltpu.async_copy(
        x_hbm.at[pl.ds(base, HALF)], x_v.at[pl.ds(0, HALF)], sem_in0)
    in1 = pltpu.async_copy(
        x_hbm.at[pl.ds(base + HALF, HALF)], x_v.at[pl.ds(HALF, HALF)],
        sem_in1)

    idx0 = lax.broadcasted_iota(jnp.int32, (L,), 0) * SUBH

    in0.wait()

    @plsc.parallel_loop(0, SUBH, unroll=8,
                        carry=jnp.zeros((L,), jnp.float32))
    def t0(j, acc):
        return acc + plsc.load_gather(x_v, [idx0 + j])

    in1.wait()

    @plsc.parallel_loop(0, SUBH, unroll=8,
                        carry=jnp.zeros((L,), jnp.float32))
    def t1(j, acc):
        return acc + plsc.load_gather(x_v, [idx0 + (HALF + j)])

    sum0 = jnp.sum(t0)
    total = sum0 + jnp.sum(t1)

    tot_v[...] = jnp.zeros((L,), jnp.float32) + total
    pltpu.sync_copy(tot_v, shared.at[pl.ds(sid * L, L)])
    plsc.subcore_barrier()
    pltpu.sync_copy(shared, all_v)

    off = jnp.zeros((L,), jnp.float32)
    for k in range(NS):
        row = all_v[pl.ds(k * L, L)]
        off = off + jnp.where(k < sid, row, jnp.zeros((L,), jnp.float32))

    @plsc.parallel_loop(0, SUBH, unroll=8, carry=off + _excl(t0))
    def _run0(j, running):
        running = running + plsc.load_gather(x_v, [idx0 + j])
        plsc.store_scatter(x_v, [idx0 + j], running)
        return running

    out0 = pltpu.async_copy(
        x_v.at[pl.ds(0, HALF)], out_hbm.at[pl.ds(base, HALF)], sem_out0)

    @plsc.parallel_loop(0, SUBH, unroll=8, carry=off + sum0 + _excl(t1))
    def _run1(j, running):
        running = running + plsc.load_gather(x_v, [idx0 + (HALF + j)])
        plsc.store_scatter(x_v, [idx0 + (HALF + j)], running)
        return running

    pltpu.sync_copy(
        x_v.at[pl.ds(HALF, HALF)], out_hbm.at[pl.ds(base + HALF, HALF)])
    out0.wait()


_sc_cumsum = pl.kernel(
    _sc_cumsum_body,
    out_type=jax.ShapeDtypeStruct((N,), jnp.float32),
    mesh=_mesh,
    compiler_params=pltpu.CompilerParams(needs_layout_passes=False),
    scratch_types=[
        pltpu.VMEM((CHUNK,), jnp.float32),
        pltpu.VMEM((L,), jnp.float32),
        pltpu.VMEM((NS * L,), jnp.float32),
        pltpu.VMEM_SHARED((NS * L,), jnp.float32),
        pltpu.SemaphoreType.DMA,
        pltpu.SemaphoreType.DMA,
        pltpu.SemaphoreType.DMA,
    ],
)


def kernel(mask_i):
    return _sc_cumsum(mask_i)


# speedup vs baseline: 1.0115x; 1.0115x over previous
import jax
import jax.numpy as jnp
from jax import lax
from jax.experimental import pallas as pl
from jax.experimental.pallas import tpu as pltpu
from jax.experimental.pallas import tpu_sc as plsc

N = 32768
NS = 16
L = 16
CHUNK = N // NS
SUB = CHUNK // L

_mesh = plsc.VectorSubcoreMesh(
    core_axis_name="c", subcore_axis_name="s", num_cores=1
)


def _sc_cumsum_body(x_hbm, out_hbm, x_v, tot_v, all_v, shared):
    sid = lax.axis_index("s")
    base = sid * CHUNK

    pltpu.sync_copy(x_hbm.at[pl.ds(base, CHUNK)], x_v)

    idx0 = lax.broadcasted_iota(jnp.int32, (L,), 0) * SUB

    @plsc.parallel_loop(0, SUB, unroll=8, carry=jnp.zeros((L,), jnp.float32))
    def lane_tot(j, acc):
        return acc + plsc.load_gather(x_v, [idx0 + j])

    total = jnp.sum(lane_tot)

    tot_v[...] = jnp.zeros((L,), jnp.float32) + total
    pltpu.sync_copy(tot_v, shared.at[pl.ds(sid * L, L)])
    plsc.subcore_barrier()
    pltpu.sync_copy(shared, all_v)

    off = jnp.zeros((L,), jnp.float32)
    for k in range(NS):
        row = all_v[pl.ds(k * L, L)]
        off = off + jnp.where(k < sid, row, jnp.zeros((L,), jnp.float32))

    lane_off = off + plsc.cumsum(lane_tot) - lane_tot

    @plsc.parallel_loop(0, SUB, unroll=8, carry=lane_off)
    def _run(j, running):
        running = running + plsc.load_gather(x_v, [idx0 + j])
        plsc.store_scatter(x_v, [idx0 + j], running)
        return running

    pltpu.sync_copy(x_v, out_hbm.at[pl.ds(base, CHUNK)])


_sc_cumsum = pl.kernel(
    _sc_cumsum_body,
    out_type=jax.ShapeDtypeStruct((N,), jnp.float32),
    mesh=_mesh,
    compiler_params=pltpu.CompilerParams(needs_layout_passes=False),
    scratch_types=[
        pltpu.VMEM((CHUNK,), jnp.float32),
        pltpu.VMEM((L,), jnp.float32),
        pltpu.VMEM((NS * L,), jnp.float32),
        pltpu.VMEM_SHARED((NS * L,), jnp.float32),
    ],
)


def kernel(mask_i):
    return _sc_cumsum(mask_i)


# speedup vs baseline: 1.0116x; 1.0001x over previous
import jax
import jax.numpy as jnp
from jax import lax
from jax.experimental import pallas as pl
from jax.experimental.pallas import tpu as pltpu
from jax.experimental.pallas import tpu_sc as plsc

N = 32768
NS = 16
L = 16
CHUNK = N // NS
SUB = CHUNK // L

_mesh = plsc.VectorSubcoreMesh(
    core_axis_name="c", subcore_axis_name="s", num_cores=1, num_subcores=NS
)


def _sc_cumsum_body(x_hbm, out_hbm, x_v, tot_v, all_v, shared):
    sid = lax.axis_index("s")
    base = sid * CHUNK

    pltpu.sync_copy(x_hbm.at[pl.ds(base, CHUNK)], x_v)

    idx0 = lax.broadcasted_iota(jnp.int32, (L,), 0) * SUB

    @plsc.parallel_loop(0, SUB, unroll=8, carry=jnp.zeros((L,), jnp.float32))
    def lane_tot(j, acc):
        return acc + plsc.load_gather(x_v, [idx0 + j])

    total = jnp.sum(lane_tot)

    tot_v[...] = jnp.zeros((L,), jnp.float32) + total
    pltpu.sync_copy(tot_v, shared.at[pl.ds(sid * L, L)])
    plsc.subcore_barrier()
    pltpu.sync_copy(shared, all_v)

    off = jnp.zeros((L,), jnp.float32)
    for k in range(NS):
        row = all_v[pl.ds(k * L, L)]
        off = off + jnp.where(k < sid, row, jnp.zeros((L,), jnp.float32))

    lane_off = off + plsc.cumsum(lane_tot) - lane_tot

    @plsc.parallel_loop(0, SUB, unroll=8, carry=lane_off)
    def _run(j, running):
        running = running + plsc.load_gather(x_v, [idx0 + j])
        plsc.store_scatter(x_v, [idx0 + j], running)
        return running

    pltpu.sync_copy(x_v, out_hbm.at[pl.ds(base, CHUNK)])


_sc_cumsum = pl.kernel(
    _sc_cumsum_body,
    out_type=jax.ShapeDtypeStruct((N,), jnp.float32),
    mesh=_mesh,
    compiler_params=pltpu.CompilerParams(needs_layout_passes=False),
    scratch_types=[
        pltpu.VMEM((CHUNK,), jnp.float32),
        pltpu.VMEM((L,), jnp.float32),
        pltpu.VMEM((NS * L,), jnp.float32),
        pltpu.VMEM_SHARED((NS * L,), jnp.float32),
    ],
)


def kernel(mask_i):
    return _sc_cumsum(mask_i)
